# Initial kernel scaffold; baseline (speedup 1.0000x reference)
import jax
import jax.numpy as jnp
from jax.experimental import pallas as pl


def kernel(x, edge_index, edge_attr, lin1_w, lin1_b, edge1_w, edge1_b, bias1, bn1_g, bn1_b, lin2_w, lin2_b, edge2_w, edge2_b, bias2, bn2_g, bn2_b):
    raise NotImplementedError("write your pallas kernel here")


# speedup vs baseline: 7.1109x; 7.1109x over previous
import functools

import jax
import jax.numpy as jnp
from jax import lax
from jax.experimental import pallas as pl
from jax.experimental.pallas import tpu as pltpu
from jax.experimental.pallas import tpu_sc as plsc

N = 10000
E = 320000
D = 128
DE = 16
H = 128

NC = 2
NS = 16
NW = NC * NS
C = 80
NCHUNKS = E // C
CPW = NCHUNKS // NW
EPW = E // NW
RPS = N // NS

_mesh = plsc.VectorSubcoreMesh(core_axis_name="c", subcore_axis_name="s")
_sc_params = pltpu.CompilerParams(use_tc_tiling_on_sc=False)


def _sc_pass1(x, src2d, dst2d, ea, zs, zt, zd, ones):
  def body(x_hbm, src_hbm, dst_hbm, ea_hbm, zs_hbm, zt_hbm, zd_hbm, ones_hbm,
           s_out, t_out, d_out,
           src_v, dst_v, rows_v, ea_v, ones_v, sem, s_sh, t_sh, d_sh):
    c = lax.axis_index("c")
    s = lax.axis_index("s")
    wid = c * NS + s

    @pl.when(c == 0)
    def _():
      pltpu.sync_copy(x_hbm.at[pl.ds(s * RPS, RPS)], s_sh.at[pl.ds(s * RPS, RPS)])

    @pl.when(c != 0)
    def _():
      pltpu.sync_copy(zs_hbm.at[pl.ds(s * RPS, RPS)], s_sh.at[pl.ds(s * RPS, RPS)])

    pltpu.sync_copy(zt_hbm.at[pl.ds(s * RPS, RPS)], t_sh.at[pl.ds(s * RPS, RPS)])
    pltpu.sync_copy(zd_hbm.at[pl.ds(s * RPS, RPS)], d_sh.at[pl.ds(s * RPS, RPS)])

    pltpu.sync_copy(src_hbm.at[pl.ds(wid * CPW, CPW)], src_v)
    pltpu.sync_copy(dst_hbm.at[pl.ds(wid * CPW, CPW)], dst_v)
    pltpu.sync_copy(ones_hbm, ones_v)
    plsc.subcore_barrier()

    def chunk(j, carry):
      pltpu.async_copy(x_hbm.at[src_v.at[j]], rows_v, sem).wait()
      pltpu.sync_copy(ea_hbm.at[pl.ds(wid * EPW + j * C, C)], ea_v)
      pltpu.sync_copy(rows_v, s_sh.at[dst_v.at[j]], add=True)
      pltpu.sync_copy(ea_v, t_sh.at[dst_v.at[j]], add=True)
      pltpu.sync_copy(ones_v, d_sh.at[dst_v.at[j]], add=True)
      return carry

    lax.fori_loop(0, CPW, chunk, 0)
    plsc.subcore_barrier()

    pltpu.sync_copy(s_sh.at[pl.ds(s * RPS, RPS)], s_out.at[c, pl.ds(s * RPS, RPS)])
    pltpu.sync_copy(t_sh.at[pl.ds(s * RPS, RPS)], t_out.at[c, pl.ds(s * RPS, RPS)])
    pltpu.sync_copy(d_sh.at[pl.ds(s * RPS, RPS)], d_out.at[c, pl.ds(s * RPS, RPS)])

  fn = pl.kernel(
      body,
      out_type=[
          jax.ShapeDtypeStruct((NC, N, D), jnp.float32),
          jax.ShapeDtypeStruct((NC, N, DE), jnp.float32),
          jax.ShapeDtypeStruct((NC, N, 8), jnp.float32),
      ],
      mesh=_mesh,
      compiler_params=_sc_params,
      scratch_types=[
          pltpu.VMEM((CPW, C), jnp.int32),
          pltpu.VMEM((CPW, C), jnp.int32),
          pltpu.VMEM((C, D), jnp.float32),
          pltpu.VMEM((C, DE), jnp.float32),
          pltpu.VMEM((C, 8), jnp.float32),
          pltpu.SemaphoreType.DMA,
          pltpu.VMEM_SHARED((N, D), jnp.float32),
          pltpu.VMEM_SHARED((N, DE), jnp.float32),
          pltpu.VMEM_SHARED((N, 8), jnp.float32),
      ],
  )
  return fn(x, src2d, dst2d, ea, zs, zt, zd, ones)


def _sc_pass2(h, src2d, dst2d, zs):
  def body(h_hbm, src_hbm, dst_hbm, zs_hbm, s_out,
           src_v, dst_v, rows_v, sem, s_sh):
    c = lax.axis_index("c")
    s = lax.axis_index("s")
    wid = c * NS + s

    @pl.when(c == 0)
    def _():
      pltpu.sync_copy(h_hbm.at[pl.ds(s * RPS, RPS)], s_sh.at[pl.ds(s * RPS, RPS)])

    @pl.when(c != 0)
    def _():
      pltpu.sync_copy(zs_hbm.at[pl.ds(s * RPS, RPS)], s_sh.at[pl.ds(s * RPS, RPS)])

    pltpu.sync_copy(src_hbm.at[pl.ds(wid * CPW, CPW)], src_v)
    pltpu.sync_copy(dst_hbm.at[pl.ds(wid * CPW, CPW)], dst_v)
    plsc.subcore_barrier()

    def chunk(j, carry):
      pltpu.async_copy(h_hbm.at[src_v.at[j]], rows_v, sem).wait()
      pltpu.sync_copy(rows_v, s_sh.at[dst_v.at[j]], add=True)
      return carry

    lax.fori_loop(0, CPW, chunk, 0)
    plsc.subcore_barrier()
    pltpu.sync_copy(s_sh.at[pl.ds(s * RPS, RPS)], s_out.at[c, pl.ds(s * RPS, RPS)])

  fn = pl.kernel(
      body,
      out_type=jax.ShapeDtypeStruct((NC, N, D), jnp.float32),
      mesh=_mesh,
      compiler_params=_sc_params,
      scratch_types=[
          pltpu.VMEM((CPW, C), jnp.int32),
          pltpu.VMEM((CPW, C), jnp.int32),
          pltpu.VMEM((C, D), jnp.float32),
          pltpu.SemaphoreType.DMA,
          pltpu.VMEM_SHARED((N, D), jnp.float32),
      ],
  )
  return fn(h, src2d, dst2d, zs)


def _tc_body(s_ref, t_ref, d_ref, lwt_ref, ewt_ref, lbeb_ref, lb_ref,
             bias_ref, g_ref, b_ref, o_ref):
  a = s_ref[0] + s_ref[1]
  tt = t_ref[0] + t_ref[1]
  deg = (d_ref[0] + d_ref[1])[:, 0:1]
  aggr = jnp.dot(a, lwt_ref[...], preferred_element_type=jnp.float32)
  aggr = aggr + jnp.dot(tt, ewt_ref[...], preferred_element_type=jnp.float32)
  aggr = aggr + deg * lbeb_ref[...] + lb_ref[...]
  r = jnp.maximum(aggr, 0.0) + bias_ref[...]
  m = jnp.mean(r, axis=0, keepdims=True)
  cen = r - m
  v = jnp.mean(cen * cen, axis=0, keepdims=True)
  o_ref[...] = cen * lax.rsqrt(v + 1e-5) * g_ref[...] + b_ref[...]


def _tc_layer(sp, tp, dp, lw, lb, ew, eb, bias, g, b):
  lwt = lw.T
  ewt = ew.T
  lbeb = (lb + eb).reshape(1, H)
  return pl.pallas_call(
      _tc_body,
      out_shape=jax.ShapeDtypeStruct((N, H), jnp.float32),
  )(sp, tp, dp, lwt, ewt, lbeb, lb.reshape(1, H), bias.reshape(1, H),
    g.reshape(1, H), b.reshape(1, H))


def kernel(x, edge_index, edge_attr, lin1_w, lin1_b, edge1_w, edge1_b, bias1,
           bn1_g, bn1_b, lin2_w, lin2_b, edge2_w, edge2_b, bias2, bn2_g, bn2_b):
  src2d = edge_index[0].astype(jnp.int32).reshape(NCHUNKS, C)
  dst2d = edge_index[1].astype(jnp.int32).reshape(NCHUNKS, C)
  zs = jnp.zeros((N, D), jnp.float32)
  zt = jnp.zeros((N, DE), jnp.float32)
  zd = jnp.zeros((N, 8), jnp.float32)
  ones = jnp.ones((C, 8), jnp.float32)

  s1p, tp, dp = _sc_pass1(x, src2d, dst2d, edge_attr, zs, zt, zd, ones)
  h1 = _tc_layer(s1p, tp, dp, lin1_w, lin1_b, edge1_w, edge1_b, bias1,
                 bn1_g, bn1_b)
  s2p = _sc_pass2(h1, src2d, dst2d, zs)
  out = _tc_layer(s2p, tp, dp, lin2_w, lin2_b, edge2_w, edge2_b, bias2,
                  bn2_g, bn2_b)
  return out


# speedup vs baseline: 9.1114x; 1.2813x over previous
import jax
import jax.numpy as jnp
from jax import lax
from jax.experimental import pallas as pl
from jax.experimental.pallas import tpu as pltpu
from jax.experimental.pallas import tpu_sc as plsc

N = 10000
E = 320000
D = 128
DE = 16
H = 128

NC = 2
NS = 16
NW = NC * NS
C = 80
NCHUNKS = E // C
CPW = NCHUNKS // NW
RPS = N // NS
NBUF = 2
EBUF = 5

_mesh = plsc.VectorSubcoreMesh(core_axis_name="c", subcore_axis_name="s")
_sc_params = pltpu.CompilerParams(use_tc_tiling_on_sc=False)


def _sc_edge_pass(dst2d, ea, zt, zd, ones):
  def body(dst_hbm, ea_hbm, zt_hbm, zd_hbm, ones_hbm, t_out, d_out,
           dst_v, ea_v, ones_v, esem0, esem1, esem2, esem3, esem4,
           t_sh, d_sh):
    c = lax.axis_index("c")
    s = lax.axis_index("s")
    wid = c * NS + s
    esems = (esem0, esem1, esem2, esem3, esem4)

    pltpu.sync_copy(zt_hbm.at[pl.ds(s * RPS, RPS)], t_sh.at[pl.ds(s * RPS, RPS)])
    pltpu.sync_copy(zd_hbm.at[pl.ds(s * RPS, RPS)], d_sh.at[pl.ds(s * RPS, RPS)])
    pltpu.sync_copy(dst_hbm.at[pl.ds(wid * CPW, CPW)], dst_v)
    pltpu.sync_copy(ones_hbm, ones_v)
    plsc.subcore_barrier()

    def outer(i, carry):
      i0 = i * EBUF
      eds = [pltpu.async_copy(ea_hbm.at[pl.ds((wid * CPW + i0 + b) * C, C)],
                              ea_v.at[b], esems[b]) for b in range(EBUF)]
      for b in range(EBUF):
        eds[b].wait()
        pltpu.sync_copy(ea_v.at[b], t_sh.at[dst_v.at[i0 + b]], add=True)
        pltpu.sync_copy(ones_v, d_sh.at[dst_v.at[i0 + b]], add=True)
      return carry

    lax.fori_loop(0, CPW // EBUF, outer, 0)
    plsc.subcore_barrier()
    pltpu.sync_copy(t_sh.at[pl.ds(s * RPS, RPS)], t_out.at[c, pl.ds(s * RPS, RPS)])
    pltpu.sync_copy(d_sh.at[pl.ds(s * RPS, RPS)], d_out.at[c, pl.ds(s * RPS, RPS)])

  fn = pl.kernel(
      body,
      out_type=[
          jax.ShapeDtypeStruct((NC, N, DE), jnp.float32),
          jax.ShapeDtypeStruct((NC, N, 8), jnp.float32),
      ],
      mesh=_mesh,
      compiler_params=_sc_params,
      scratch_types=[
          pltpu.VMEM((CPW, C), jnp.int32),
          pltpu.VMEM((EBUF, C, DE), jnp.float32),
          pltpu.VMEM((C, 8), jnp.float32),
          pltpu.SemaphoreType.DMA,
          pltpu.SemaphoreType.DMA,
          pltpu.SemaphoreType.DMA,
          pltpu.SemaphoreType.DMA,
          pltpu.SemaphoreType.DMA,
          pltpu.VMEM_SHARED((N, DE), jnp.float32),
          pltpu.VMEM_SHARED((N, 8), jnp.float32),
      ],
  )
  return fn(dst2d, ea, zt, zd, ones)


def _sc_spmm(tbl, src2d, dst2d, zs):
  def body(tbl_hbm, src_hbm, dst_hbm, zs_hbm, s_out,
           src_v, dst_v, rows_v, rsem0, rsem1, s_sh):
    c = lax.axis_index("c")
    s = lax.axis_index("s")
    wid = c * NS + s
    rsems = (rsem0, rsem1)

    @pl.when(c == 0)
    def _():
      pltpu.sync_copy(tbl_hbm.at[pl.ds(s * RPS, RPS)], s_sh.at[pl.ds(s * RPS, RPS)])

    @pl.when(c != 0)
    def _():
      pltpu.sync_copy(zs_hbm.at[pl.ds(s * RPS, RPS)], s_sh.at[pl.ds(s * RPS, RPS)])

    pltpu.sync_copy(src_hbm.at[pl.ds(wid * CPW, CPW)], src_v)
    pltpu.sync_copy(dst_hbm.at[pl.ds(wid * CPW, CPW)], dst_v)
    plsc.subcore_barrier()

    def outer(i, carry):
      i0 = i * NBUF
      rds = [pltpu.async_copy(tbl_hbm.at[src_v.at[i0 + b]], rows_v.at[b],
                              rsems[b]) for b in range(NBUF)]
      for b in range(NBUF):
        rds[b].wait()
        pltpu.sync_copy(rows_v.at[b], s_sh.at[dst_v.at[i0 + b]], add=True)
      return carry

    lax.fori_loop(0, CPW // NBUF, outer, 0)
    tail = CPW - CPW % NBUF
    for b in range(CPW % NBUF):
      rd = pltpu.async_copy(tbl_hbm.at[src_v.at[tail + b]], rows_v.at[b],
                            rsems[b])
      rd.wait()
      pltpu.sync_copy(rows_v.at[b], s_sh.at[dst_v.at[tail + b]], add=True)

    plsc.subcore_barrier()
    pltpu.sync_copy(s_sh.at[pl.ds(s * RPS, RPS)], s_out.at[c, pl.ds(s * RPS, RPS)])

  fn = pl.kernel(
      body,
      out_type=jax.ShapeDtypeStruct((NC, N, D), jnp.float32),
      mesh=_mesh,
      compiler_params=_sc_params,
      scratch_types=[
          pltpu.VMEM((CPW, C), jnp.int32),
          pltpu.VMEM((CPW, C), jnp.int32),
          pltpu.VMEM((NBUF, C, D), jnp.float32),
          pltpu.SemaphoreType.DMA,
          pltpu.SemaphoreType.DMA,
          pltpu.VMEM_SHARED((N, D), jnp.float32),
      ],
  )
  return fn(tbl, src2d, dst2d, zs)


def _tc_body(s_ref, t_ref, d_ref, lwt_ref, ewt_ref, lbeb_ref, lb_ref,
             bias_ref, g_ref, b_ref, o_ref):
  a = s_ref[0] + s_ref[1]
  tt = t_ref[0] + t_ref[1]
  deg = (d_ref[0] + d_ref[1])[:, 0:1]
  aggr = jnp.dot(a, lwt_ref[...], preferred_element_type=jnp.float32)
  aggr = aggr + jnp.dot(tt, ewt_ref[...], preferred_element_type=jnp.float32)
  aggr = aggr + deg * lbeb_ref[...] + lb_ref[...]
  r = jnp.maximum(aggr, 0.0) + bias_ref[...]
  m = jnp.mean(r, axis=0, keepdims=True)
  cen = r - m
  v = jnp.mean(cen * cen, axis=0, keepdims=True)
  o_ref[...] = cen * lax.rsqrt(v + 1e-5) * g_ref[...] + b_ref[...]


def _tc_layer(sp, tp, dp, lw, lb, ew, eb, bias, g, b):
  lwt = lw.T
  ewt = ew.T
  lbeb = (lb + eb).reshape(1, H)
  return pl.pallas_call(
      _tc_body,
      out_shape=jax.ShapeDtypeStruct((N, H), jnp.float32),
  )(sp, tp, dp, lwt, ewt, lbeb, lb.reshape(1, H), bias.reshape(1, H),
    g.reshape(1, H), b.reshape(1, H))


def kernel(x, edge_index, edge_attr, lin1_w, lin1_b, edge1_w, edge1_b, bias1,
           bn1_g, bn1_b, lin2_w, lin2_b, edge2_w, edge2_b, bias2, bn2_g, bn2_b):
  src2d = edge_index[0].astype(jnp.int32).reshape(NCHUNKS, C)
  dst2d = edge_index[1].astype(jnp.int32).reshape(NCHUNKS, C)
  zs = jnp.zeros((N, D), jnp.float32)
  zt = jnp.zeros((N, DE), jnp.float32)
  zd = jnp.zeros((N, 8), jnp.float32)
  ones = jnp.ones((C, 8), jnp.float32)

  tp, dp = _sc_edge_pass(dst2d, edge_attr, zt, zd, ones)
  s1p = _sc_spmm(x, src2d, dst2d, zs)
  h1 = _tc_layer(s1p, tp, dp, lin1_w, lin1_b, edge1_w, edge1_b, bias1,
                 bn1_g, bn1_b)
  s2p = _sc_spmm(h1, src2d, dst2d, zs)
  out = _tc_layer(s2p, tp, dp, lin2_w, lin2_b, edge2_w, edge2_b, bias2,
                  bn2_g, bn2_b)
  return out


# speedup vs baseline: 9.4299x; 1.0350x over previous
import jax
import jax.numpy as jnp
from jax import lax
from jax.experimental import pallas as pl
from jax.experimental.pallas import tpu as pltpu
from jax.experimental.pallas import tpu_sc as plsc

N = 10000
E = 320000
D = 128
DE = 16
H = 128

NC = 2
NS = 16
NW = NC * NS
C = 80
NCHUNKS = E // C
CPW = NCHUNKS // NW
RPS = N // NS
NBUF = 3
EBUF = 5

_mesh = plsc.VectorSubcoreMesh(core_axis_name="c", subcore_axis_name="s")
_sc_params = pltpu.CompilerParams(use_tc_tiling_on_sc=False)


def _sc_edge_pass(dst2d, ea, zt, zd, ones):
  def body(dst_hbm, ea_hbm, zt_hbm, zd_hbm, ones_hbm, t_out, d_out,
           dst_v, ea_v, ones_v, esem0, esem1, esem2, esem3, esem4,
           t_sh, d_sh):
    c = lax.axis_index("c")
    s = lax.axis_index("s")
    wid = c * NS + s
    esems = (esem0, esem1, esem2, esem3, esem4)

    pltpu.sync_copy(zt_hbm.at[pl.ds(s * RPS, RPS)], t_sh.at[pl.ds(s * RPS, RPS)])
    pltpu.sync_copy(zd_hbm.at[pl.ds(s * RPS, RPS)], d_sh.at[pl.ds(s * RPS, RPS)])
    pltpu.sync_copy(dst_hbm.at[pl.ds(wid * CPW, CPW)], dst_v)
    pltpu.sync_copy(ones_hbm, ones_v)
    plsc.subcore_barrier()

    def outer(i, carry):
      i0 = i * EBUF
      eds = [pltpu.async_copy(ea_hbm.at[pl.ds((wid * CPW + i0 + b) * C, C)],
                              ea_v.at[b], esems[b]) for b in range(EBUF)]
      for b in range(EBUF):
        eds[b].wait()
        pltpu.sync_copy(ea_v.at[b], t_sh.at[dst_v.at[i0 + b]], add=True)
        pltpu.sync_copy(ones_v, d_sh.at[dst_v.at[i0 + b]], add=True)
      return carry

    lax.fori_loop(0, CPW // EBUF, outer, 0)
    plsc.subcore_barrier()
    pltpu.sync_copy(t_sh.at[pl.ds(s * RPS, RPS)], t_out.at[c, pl.ds(s * RPS, RPS)])
    pltpu.sync_copy(d_sh.at[pl.ds(s * RPS, RPS)], d_out.at[c, pl.ds(s * RPS, RPS)])

  fn = pl.kernel(
      body,
      out_type=[
          jax.ShapeDtypeStruct((NC, N, DE), jnp.float32),
          jax.ShapeDtypeStruct((NC, N, 8), jnp.float32),
      ],
      mesh=_mesh,
      compiler_params=_sc_params,
      scratch_types=[
          pltpu.VMEM((CPW, C), jnp.int32),
          pltpu.VMEM((EBUF, C, DE), jnp.float32),
          pltpu.VMEM((C, 8), jnp.float32),
          pltpu.SemaphoreType.DMA,
          pltpu.SemaphoreType.DMA,
          pltpu.SemaphoreType.DMA,
          pltpu.SemaphoreType.DMA,
          pltpu.SemaphoreType.DMA,
          pltpu.VMEM_SHARED((N, DE), jnp.float32),
          pltpu.VMEM_SHARED((N, 8), jnp.float32),
      ],
  )
  return fn(dst2d, ea, zt, zd, ones)


def _sc_spmm(tbl, src2d, dst2d, zs):
  def body(tbl_hbm, src_hbm, dst_hbm, zs_hbm, s_out,
           src_v, dst_v, rows_v, rsem0, rsem1, rsem2, dsem, s_sh):
    c = lax.axis_index("c")
    s = lax.axis_index("s")
    wid = c * NS + s
    rsems = (rsem0, rsem1, rsem2)

    @pl.when(c == 0)
    def _():
      pltpu.sync_copy(tbl_hbm.at[pl.ds(s * RPS, RPS)], s_sh.at[pl.ds(s * RPS, RPS)])

    @pl.when(c != 0)
    def _():
      pltpu.sync_copy(zs_hbm.at[pl.ds(s * RPS, RPS)], s_sh.at[pl.ds(s * RPS, RPS)])

    pltpu.sync_copy(src_hbm.at[pl.ds(wid * CPW, CPW)], src_v)
    plsc.subcore_barrier()

    def batch(i0, nb):
      dd = pltpu.async_copy(dst_hbm.at[pl.ds(wid * CPW + i0, nb)],
                            dst_v.at[pl.ds(0, nb)], dsem)
      rds = [pltpu.async_copy(tbl_hbm.at[src_v.at[i0 + b]], rows_v.at[b],
                              rsems[b]) for b in range(nb)]
      dd.wait()
      for b in range(nb):
        rds[b].wait()
        pltpu.sync_copy(rows_v.at[b], s_sh.at[dst_v.at[b]], add=True)

    def outer(i, carry):
      batch(i * NBUF, NBUF)
      return carry

    lax.fori_loop(0, CPW // NBUF, outer, 0)
    if CPW % NBUF:
      batch(CPW - CPW % NBUF, CPW % NBUF)

    plsc.subcore_barrier()
    pltpu.sync_copy(s_sh.at[pl.ds(s * RPS, RPS)], s_out.at[c, pl.ds(s * RPS, RPS)])

  fn = pl.kernel(
      body,
      out_type=jax.ShapeDtypeStruct((NC, N, D), jnp.float32),
      mesh=_mesh,
      compiler_params=_sc_params,
      scratch_types=[
          pltpu.VMEM((CPW, C), jnp.int32),
          pltpu.VMEM((NBUF, C), jnp.int32),
          pltpu.VMEM((NBUF, C, D), jnp.float32),
          pltpu.SemaphoreType.DMA,
          pltpu.SemaphoreType.DMA,
          pltpu.SemaphoreType.DMA,
          pltpu.SemaphoreType.DMA,
          pltpu.VMEM_SHARED((N, D), jnp.float32),
      ],
  )
  return fn(tbl, src2d, dst2d, zs)


def _tc_body(s_ref, t_ref, d_ref, lwt_ref, ewt_ref, lbeb_ref, lb_ref,
             bias_ref, g_ref, b_ref, o_ref):
  a = s_ref[0] + s_ref[1]
  tt = t_ref[0] + t_ref[1]
  deg = (d_ref[0] + d_ref[1])[:, 0:1]
  aggr = jnp.dot(a, lwt_ref[...], preferred_element_type=jnp.float32)
  aggr = aggr + jnp.dot(tt, ewt_ref[...], preferred_element_type=jnp.float32)
  aggr = aggr + deg * lbeb_ref[...] + lb_ref[...]
  r = jnp.maximum(aggr, 0.0) + bias_ref[...]
  m = jnp.mean(r, axis=0, keepdims=True)
  cen = r - m
  v = jnp.mean(cen * cen, axis=0, keepdims=True)
  o_ref[...] = cen * lax.rsqrt(v + 1e-5) * g_ref[...] + b_ref[...]


def _tc_layer(sp, tp, dp, lw, lb, ew, eb, bias, g, b):
  lwt = lw.T
  ewt = ew.T
  lbeb = (lb + eb).reshape(1, H)
  return pl.pallas_call(
      _tc_body,
      out_shape=jax.ShapeDtypeStruct((N, H), jnp.float32),
  )(sp, tp, dp, lwt, ewt, lbeb, lb.reshape(1, H), bias.reshape(1, H),
    g.reshape(1, H), b.reshape(1, H))


def kernel(x, edge_index, edge_attr, lin1_w, lin1_b, edge1_w, edge1_b, bias1,
           bn1_g, bn1_b, lin2_w, lin2_b, edge2_w, edge2_b, bias2, bn2_g, bn2_b):
  src2d = edge_index[0].astype(jnp.int32).reshape(NCHUNKS, C)
  dst2d = edge_index[1].astype(jnp.int32).reshape(NCHUNKS, C)
  zs = jnp.zeros((N, D), jnp.float32)
  zt = jnp.zeros((N, DE), jnp.float32)
  zd = jnp.zeros((N, 8), jnp.float32)
  ones = jnp.ones((C, 8), jnp.float32)

  tp, dp = _sc_edge_pass(dst2d, edge_attr, zt, zd, ones)
  s1p = _sc_spmm(x, src2d, dst2d, zs)
  h1 = _tc_layer(s1p, tp, dp, lin1_w, lin1_b, edge1_w, edge1_b, bias1,
                 bn1_g, bn1_b)
  s2p = _sc_spmm(h1, src2d, dst2d, zs)
  out = _tc_layer(s2p, tp, dp, lin2_w, lin2_b, edge2_w, edge2_b, bias2,
                  bn2_g, bn2_b)
  return out


# speedup vs baseline: 9.5925x; 1.0172x over previous
import jax
import jax.numpy as jnp
from jax import lax
from jax.experimental import pallas as pl
from jax.experimental.pallas import tpu as pltpu
from jax.experimental.pallas import tpu_sc as plsc

N = 10000
E = 320000
D = 128
DE = 16
H = 128

NC = 2
NS = 16
NW = NC * NS
C = 80
NCHUNKS = E // C
CPW = NCHUNKS // NW
RPS = N // NS
NBUF = 3
EBUF = 5

_mesh = plsc.VectorSubcoreMesh(core_axis_name="c", subcore_axis_name="s")
_sc_params = pltpu.CompilerParams(use_tc_tiling_on_sc=False)


def _sc_edge_pass(dst2d, ea, zt, zd, ones):
  def body(dst_hbm, ea_hbm, zt_hbm, zd_hbm, ones_hbm, t_out, d_out,
           dst_v, ea_v, ones_v, esem0, esem1, esem2, esem3, esem4,
           t_sh, d_sh):
    c = lax.axis_index("c")
    s = lax.axis_index("s")
    wid = c * NS + s
    esems = (esem0, esem1, esem2, esem3, esem4)

    pltpu.sync_copy(zt_hbm.at[pl.ds(s * RPS, RPS)], t_sh.at[pl.ds(s * RPS, RPS)])
    pltpu.sync_copy(zd_hbm.at[pl.ds(s * RPS, RPS)], d_sh.at[pl.ds(s * RPS, RPS)])
    pltpu.sync_copy(dst_hbm.at[pl.ds(wid * CPW, CPW)], dst_v)
    pltpu.sync_copy(ones_hbm, ones_v)
    plsc.subcore_barrier()

    def outer(i, carry):
      i0 = i * EBUF
      eds = [pltpu.async_copy(ea_hbm.at[pl.ds((wid * CPW + i0 + b) * C, C)],
                              ea_v.at[b], esems[b]) for b in range(EBUF)]
      for b in range(EBUF):
        eds[b].wait()
        pltpu.sync_copy(ea_v.at[b], t_sh.at[dst_v.at[i0 + b]], add=True)
        pltpu.sync_copy(ones_v, d_sh.at[dst_v.at[i0 + b]], add=True)
      return carry

    lax.fori_loop(0, CPW // EBUF, outer, 0)
    plsc.subcore_barrier()
    pltpu.sync_copy(t_sh.at[pl.ds(s * RPS, RPS)], t_out.at[c, pl.ds(s * RPS, RPS)])
    pltpu.sync_copy(d_sh.at[pl.ds(s * RPS, RPS)], d_out.at[c, pl.ds(s * RPS, RPS)])

  fn = pl.kernel(
      body,
      out_type=[
          jax.ShapeDtypeStruct((NC, N, DE), jnp.float32),
          jax.ShapeDtypeStruct((NC, N, 8), jnp.float32),
      ],
      mesh=_mesh,
      compiler_params=_sc_params,
      scratch_types=[
          pltpu.VMEM((CPW, C), jnp.int32),
          pltpu.VMEM((EBUF, C, DE), jnp.float32),
          pltpu.VMEM((C, 8), jnp.float32),
          pltpu.SemaphoreType.DMA,
          pltpu.SemaphoreType.DMA,
          pltpu.SemaphoreType.DMA,
          pltpu.SemaphoreType.DMA,
          pltpu.SemaphoreType.DMA,
          pltpu.VMEM_SHARED((N, DE), jnp.float32),
          pltpu.VMEM_SHARED((N, 8), jnp.float32),
      ],
  )
  return fn(dst2d, ea, zt, zd, ones)


def _sc_spmm(tbl, src2d, dst2d, zs):
  def body(tbl_hbm, src_hbm, dst_hbm, zs_hbm, s_out,
           src_v, dst_v, rows_v, rsem0, rsem1, rsem2,
           ssem0, ssem1, ssem2, dsem, s_sh):
    c = lax.axis_index("c")
    s = lax.axis_index("s")
    wid = c * NS + s
    rsems = (rsem0, rsem1, rsem2)
    ssems = (ssem0, ssem1, ssem2)

    @pl.when(c == 0)
    def _():
      pltpu.sync_copy(tbl_hbm.at[pl.ds(s * RPS, RPS)], s_sh.at[pl.ds(s * RPS, RPS)])

    @pl.when(c != 0)
    def _():
      pltpu.sync_copy(zs_hbm.at[pl.ds(s * RPS, RPS)], s_sh.at[pl.ds(s * RPS, RPS)])

    pltpu.sync_copy(src_hbm.at[pl.ds(wid * CPW, CPW)], src_v)
    plsc.subcore_barrier()

    def batch(i0, nb):
      dd = pltpu.async_copy(dst_hbm.at[pl.ds(wid * CPW + i0, nb)],
                            dst_v.at[pl.ds(0, nb)], dsem)
      rds = [pltpu.async_copy(tbl_hbm.at[src_v.at[i0 + b]], rows_v.at[b],
                              rsems[b]) for b in range(nb)]
      dd.wait()
      sds = []
      for b in range(nb):
        rds[b].wait()
        sds.append(pltpu.async_copy(rows_v.at[b], s_sh.at[dst_v.at[b]],
                                    ssems[b], add=True))
      for sd in sds:
        sd.wait()

    def outer(i, carry):
      batch(i * NBUF, NBUF)
      return carry

    lax.fori_loop(0, CPW // NBUF, outer, 0)
    if CPW % NBUF:
      batch(CPW - CPW % NBUF, CPW % NBUF)

    plsc.subcore_barrier()
    pltpu.sync_copy(s_sh.at[pl.ds(s * RPS, RPS)], s_out.at[c, pl.ds(s * RPS, RPS)])

  fn = pl.kernel(
      body,
      out_type=jax.ShapeDtypeStruct((NC, N, D), jnp.float32),
      mesh=_mesh,
      compiler_params=_sc_params,
      scratch_types=[
          pltpu.VMEM((CPW, C), jnp.int32),
          pltpu.VMEM((NBUF, C), jnp.int32),
          pltpu.VMEM((NBUF, C, D), jnp.float32),
          pltpu.SemaphoreType.DMA,
          pltpu.SemaphoreType.DMA,
          pltpu.SemaphoreType.DMA,
          pltpu.SemaphoreType.DMA,
          pltpu.SemaphoreType.DMA,
          pltpu.SemaphoreType.DMA,
          pltpu.SemaphoreType.DMA,
          pltpu.VMEM_SHARED((N, D), jnp.float32),
      ],
  )
  return fn(tbl, src2d, dst2d, zs)


def _tc_body(s_ref, t_ref, d_ref, lwt_ref, ewt_ref, lbeb_ref, lb_ref,
             bias_ref, g_ref, b_ref, o_ref):
  a = s_ref[0] + s_ref[1]
  tt = t_ref[0] + t_ref[1]
  deg = (d_ref[0] + d_ref[1])[:, 0:1]
  aggr = jnp.dot(a, lwt_ref[...], preferred_element_type=jnp.float32)
  aggr = aggr + jnp.dot(tt, ewt_ref[...], preferred_element_type=jnp.float32)
  aggr = aggr + deg * lbeb_ref[...] + lb_ref[...]
  r = jnp.maximum(aggr, 0.0) + bias_ref[...]
  m = jnp.mean(r, axis=0, keepdims=True)
  cen = r - m
  v = jnp.mean(cen * cen, axis=0, keepdims=True)
  o_ref[...] = cen * lax.rsqrt(v + 1e-5) * g_ref[...] + b_ref[...]


def _tc_layer(sp, tp, dp, lw, lb, ew, eb, bias, g, b):
  lwt = lw.T
  ewt = ew.T
  lbeb = (lb + eb).reshape(1, H)
  return pl.pallas_call(
      _tc_body,
      out_shape=jax.ShapeDtypeStruct((N, H), jnp.float32),
  )(sp, tp, dp, lwt, ewt, lbeb, lb.reshape(1, H), bias.reshape(1, H),
    g.reshape(1, H), b.reshape(1, H))


def kernel(x, edge_index, edge_attr, lin1_w, lin1_b, edge1_w, edge1_b, bias1,
           bn1_g, bn1_b, lin2_w, lin2_b, edge2_w, edge2_b, bias2, bn2_g, bn2_b):
  src2d = edge_index[0].astype(jnp.int32).reshape(NCHUNKS, C)
  dst2d = edge_index[1].astype(jnp.int32).reshape(NCHUNKS, C)
  zs = jnp.zeros((N, D), jnp.float32)
  zt = jnp.zeros((N, DE), jnp.float32)
  zd = jnp.zeros((N, 8), jnp.float32)
  ones = jnp.ones((C, 8), jnp.float32)

  tp, dp = _sc_edge_pass(dst2d, edge_attr, zt, zd, ones)
  s1p = _sc_spmm(x, src2d, dst2d, zs)
  h1 = _tc_layer(s1p, tp, dp, lin1_w, lin1_b, edge1_w, edge1_b, bias1,
                 bn1_g, bn1_b)
  s2p = _sc_spmm(h1, src2d, dst2d, zs)
  out = _tc_layer(s2p, tp, dp, lin2_w, lin2_b, edge2_w, edge2_b, bias2,
                  bn2_g, bn2_b)
  return out


# speedup vs baseline: 9.6177x; 1.0026x over previous
import jax
import jax.numpy as jnp
from jax import lax
from jax.experimental import pallas as pl
from jax.experimental.pallas import tpu as pltpu
from jax.experimental.pallas import tpu_sc as plsc

N = 10000
E = 320000
D = 128
DE = 16
H = 128

NC = 2
NS = 16
NW = NC * NS
C = 80
NCHUNKS = E // C
CPW = NCHUNKS // NW
RPS = N // NS
NBUF = 3
EBUF = 5

_mesh = plsc.VectorSubcoreMesh(core_axis_name="c", subcore_axis_name="s")
_sc_params = pltpu.CompilerParams(use_tc_tiling_on_sc=False)


MB = 2


def _sc_pass1(x, src2d, dst2d, ea, zs, zt, zd, ones):
  def body(x_hbm, src_hbm, dst_hbm, ea_hbm, zs_hbm, zt_hbm, zd_hbm, ones_hbm,
           s_out, t_out, d_out,
           src_v, dst_v, rows_v, ea_v, ones_v,
           rsem0, rsem1, ssem0, ssem1, esem0, esem1, tsem0, tsem1, dsem,
           s_sh, t_sh, d_sh):
    c = lax.axis_index("c")
    s = lax.axis_index("s")
    wid = c * NS + s
    rsems = (rsem0, rsem1)
    ssems = (ssem0, ssem1)
    esems = (esem0, esem1)
    tsems = (tsem0, tsem1)

    @pl.when(c == 0)
    def _():
      pltpu.sync_copy(x_hbm.at[pl.ds(s * RPS, RPS)], s_sh.at[pl.ds(s * RPS, RPS)])

    @pl.when(c != 0)
    def _():
      pltpu.sync_copy(zs_hbm.at[pl.ds(s * RPS, RPS)], s_sh.at[pl.ds(s * RPS, RPS)])

    pltpu.sync_copy(zt_hbm.at[pl.ds(s * RPS, RPS)], t_sh.at[pl.ds(s * RPS, RPS)])
    pltpu.sync_copy(zd_hbm.at[pl.ds(s * RPS, RPS)], d_sh.at[pl.ds(s * RPS, RPS)])
    pltpu.sync_copy(src_hbm.at[pl.ds(wid * CPW, CPW)], src_v)
    pltpu.sync_copy(ones_hbm, ones_v)
    plsc.subcore_barrier()

    def batch(i0, nb):
      dd = pltpu.async_copy(dst_hbm.at[pl.ds(wid * CPW + i0, nb)],
                            dst_v.at[pl.ds(0, nb)], dsem)
      rds = [pltpu.async_copy(x_hbm.at[src_v.at[i0 + b]], rows_v.at[b],
                              rsems[b]) for b in range(nb)]
      eds = [pltpu.async_copy(ea_hbm.at[pl.ds((wid * CPW + i0 + b) * C, C)],
                              ea_v.at[b], esems[b]) for b in range(nb)]
      dd.wait()
      sds = []
      for b in range(nb):
        rds[b].wait()
        sds.append(pltpu.async_copy(rows_v.at[b], s_sh.at[dst_v.at[b]],
                                    ssems[b], add=True))
        eds[b].wait()
        sds.append(pltpu.async_copy(ea_v.at[b], t_sh.at[dst_v.at[b]],
                                    tsems[b], add=True))
        pltpu.sync_copy(ones_v, d_sh.at[dst_v.at[b]], add=True)
      for sd in sds:
        sd.wait()

    def outer(i, carry):
      batch(i * MB, MB)
      return carry

    lax.fori_loop(0, CPW // MB, outer, 0)
    if CPW % MB:
      batch(CPW - CPW % MB, CPW % MB)

    plsc.subcore_barrier()
    pltpu.sync_copy(s_sh.at[pl.ds(s * RPS, RPS)], s_out.at[c, pl.ds(s * RPS, RPS)])
    pltpu.sync_copy(t_sh.at[pl.ds(s * RPS, RPS)], t_out.at[c, pl.ds(s * RPS, RPS)])
    pltpu.sync_copy(d_sh.at[pl.ds(s * RPS, RPS)], d_out.at[c, pl.ds(s * RPS, RPS)])

  fn = pl.kernel(
      body,
      out_type=[
          jax.ShapeDtypeStruct((NC, N, D), jnp.float32),
          jax.ShapeDtypeStruct((NC, N, DE), jnp.float32),
          jax.ShapeDtypeStruct((NC, N, 8), jnp.float32),
      ],
      mesh=_mesh,
      compiler_params=_sc_params,
      scratch_types=[
          pltpu.VMEM((CPW, C), jnp.int32),
          pltpu.VMEM((MB, C), jnp.int32),
          pltpu.VMEM((MB, C, D), jnp.float32),
          pltpu.VMEM((MB, C, DE), jnp.float32),
          pltpu.VMEM((C, 8), jnp.float32),
          pltpu.SemaphoreType.DMA,
          pltpu.SemaphoreType.DMA,
          pltpu.SemaphoreType.DMA,
          pltpu.SemaphoreType.DMA,
          pltpu.SemaphoreType.DMA,
          pltpu.SemaphoreType.DMA,
          pltpu.SemaphoreType.DMA,
          pltpu.SemaphoreType.DMA,
          pltpu.SemaphoreType.DMA,
          pltpu.VMEM_SHARED((N, D), jnp.float32),
          pltpu.VMEM_SHARED((N, DE), jnp.float32),
          pltpu.VMEM_SHARED((N, 8), jnp.float32),
      ],
  )
  return fn(x, src2d, dst2d, ea, zs, zt, zd, ones)


def _sc_spmm(tbl, src2d, dst2d, zs):
  def body(tbl_hbm, src_hbm, dst_hbm, zs_hbm, s_out,
           src_v, dst_v, rows_v, rsem0, rsem1, rsem2,
           ssem0, ssem1, ssem2, dsem, s_sh):
    c = lax.axis_index("c")
    s = lax.axis_index("s")
    wid = c * NS + s
    rsems = (rsem0, rsem1, rsem2)
    ssems = (ssem0, ssem1, ssem2)

    @pl.when(c == 0)
    def _():
      pltpu.sync_copy(tbl_hbm.at[pl.ds(s * RPS, RPS)], s_sh.at[pl.ds(s * RPS, RPS)])

    @pl.when(c != 0)
    def _():
      pltpu.sync_copy(zs_hbm.at[pl.ds(s * RPS, RPS)], s_sh.at[pl.ds(s * RPS, RPS)])

    pltpu.sync_copy(src_hbm.at[pl.ds(wid * CPW, CPW)], src_v)
    plsc.subcore_barrier()

    def batch(i0, nb):
      dd = pltpu.async_copy(dst_hbm.at[pl.ds(wid * CPW + i0, nb)],
                            dst_v.at[pl.ds(0, nb)], dsem)
      rds = [pltpu.async_copy(tbl_hbm.at[src_v.at[i0 + b]], rows_v.at[b],
                              rsems[b]) for b in range(nb)]
      dd.wait()
      sds = []
      for b in range(nb):
        rds[b].wait()
        sds.append(pltpu.async_copy(rows_v.at[b], s_sh.at[dst_v.at[b]],
                                    ssems[b], add=True))
      for sd in sds:
        sd.wait()

    def outer(i, carry):
      batch(i * NBUF, NBUF)
      return carry

    lax.fori_loop(0, CPW // NBUF, outer, 0)
    if CPW % NBUF:
      batch(CPW - CPW % NBUF, CPW % NBUF)

    plsc.subcore_barrier()
    pltpu.sync_copy(s_sh.at[pl.ds(s * RPS, RPS)], s_out.at[c, pl.ds(s * RPS, RPS)])

  fn = pl.kernel(
      body,
      out_type=jax.ShapeDtypeStruct((NC, N, D), jnp.float32),
      mesh=_mesh,
      compiler_params=_sc_params,
      scratch_types=[
          pltpu.VMEM((CPW, C), jnp.int32),
          pltpu.VMEM((NBUF, C), jnp.int32),
          pltpu.VMEM((NBUF, C, D), jnp.float32),
          pltpu.SemaphoreType.DMA,
          pltpu.SemaphoreType.DMA,
          pltpu.SemaphoreType.DMA,
          pltpu.SemaphoreType.DMA,
          pltpu.SemaphoreType.DMA,
          pltpu.SemaphoreType.DMA,
          pltpu.SemaphoreType.DMA,
          pltpu.VMEM_SHARED((N, D), jnp.float32),
      ],
  )
  return fn(tbl, src2d, dst2d, zs)


def _tc_body(s_ref, t_ref, d_ref, lwt_ref, ewt_ref, lbeb_ref, lb_ref,
             bias_ref, g_ref, b_ref, o_ref):
  a = s_ref[0] + s_ref[1]
  tt = t_ref[0] + t_ref[1]
  deg = (d_ref[0] + d_ref[1])[:, 0:1]
  aggr = jnp.dot(a, lwt_ref[...], preferred_element_type=jnp.float32)
  aggr = aggr + jnp.dot(tt, ewt_ref[...], preferred_element_type=jnp.float32)
  aggr = aggr + deg * lbeb_ref[...] + lb_ref[...]
  r = jnp.maximum(aggr, 0.0) + bias_ref[...]
  m = jnp.mean(r, axis=0, keepdims=True)
  cen = r - m
  v = jnp.mean(cen * cen, axis=0, keepdims=True)
  o_ref[...] = cen * lax.rsqrt(v + 1e-5) * g_ref[...] + b_ref[...]


def _tc_layer(sp, tp, dp, lw, lb, ew, eb, bias, g, b):
  lwt = lw.T
  ewt = ew.T
  lbeb = (lb + eb).reshape(1, H)
  return pl.pallas_call(
      _tc_body,
      out_shape=jax.ShapeDtypeStruct((N, H), jnp.float32),
  )(sp, tp, dp, lwt, ewt, lbeb, lb.reshape(1, H), bias.reshape(1, H),
    g.reshape(1, H), b.reshape(1, H))


def kernel(x, edge_index, edge_attr, lin1_w, lin1_b, edge1_w, edge1_b, bias1,
           bn1_g, bn1_b, lin2_w, lin2_b, edge2_w, edge2_b, bias2, bn2_g, bn2_b):
  src2d = edge_index[0].astype(jnp.int32).reshape(NCHUNKS, C)
  dst2d = edge_index[1].astype(jnp.int32).reshape(NCHUNKS, C)
  zs = jnp.zeros((N, D), jnp.float32)
  zt = jnp.zeros((N, DE), jnp.float32)
  zd = jnp.zeros((N, 8), jnp.float32)
  ones = jnp.ones((C, 8), jnp.float32)

  s1p, tp, dp = _sc_pass1(x, src2d, dst2d, edge_attr, zs, zt, zd, ones)
  h1 = _tc_layer(s1p, tp, dp, lin1_w, lin1_b, edge1_w, edge1_b, bias1,
                 bn1_g, bn1_b)
  s2p = _sc_spmm(h1, src2d, dst2d, zs)
  out = _tc_layer(s2p, tp, dp, lin2_w, lin2_b, edge2_w, edge2_b, bias2,
                  bn2_g, bn2_b)
  return out


# speedup vs baseline: 9.6985x; 1.0084x over previous
import jax
import jax.numpy as jnp
from jax import lax
from jax.experimental import pallas as pl
from jax.experimental.pallas import tpu as pltpu
from jax.experimental.pallas import tpu_sc as plsc

N = 10000
E = 320000
D = 128
DE = 16
H = 128

NC = 2
NS = 16
NW = NC * NS
C = 80
NCHUNKS = E // C
CPW = NCHUNKS // NW
RPS = N // NS
NBUF = 3
EBUF = 5

_mesh = plsc.VectorSubcoreMesh(core_axis_name="c", subcore_axis_name="s")
_sc_params = pltpu.CompilerParams(use_tc_tiling_on_sc=False)


MB = 2


EPW = E // NW


def _zero_rows(buf, nrows):
  zv = jnp.zeros((16,), jnp.float32)

  def zrow(r, carry):
    for k in range(8):
      buf[r, pl.ds(k * 16, 16)] = zv
    return carry

  lax.fori_loop(0, nrows, zrow, 0)


def _zero_stripe(sh, base, zsrc, width):
  for k in range(RPS // C):
    pltpu.sync_copy(zsrc, sh.at[pl.ds(base + k * C, C)])
  rem = RPS % C
  if rem:
    pltpu.sync_copy(zsrc.at[pl.ds(0, rem)],
                    sh.at[pl.ds(base + (RPS // C) * C, rem)])


def _sc_pass1(x, ei, ea, zd, ones):
  def body(x_hbm, ei_hbm, ea_hbm, zd_hbm, ones_hbm,
           s_out, t_out, d_out,
           src_v, dst_v, rows_v, ea_v, ones_v,
           rsem0, rsem1, ssem0, ssem1, esem0, esem1, tsem0, tsem1, dsem,
           s_sh, t_sh, d_sh):
    c = lax.axis_index("c")
    s = lax.axis_index("s")
    wid = c * NS + s
    rsems = (rsem0, rsem1)
    ssems = (ssem0, ssem1)
    esems = (esem0, esem1)
    tsems = (tsem0, tsem1)

    _zero_rows(rows_v.at[0], C)

    def zear(r, carry):
      ea_v[0, r, :] = jnp.zeros((16,), jnp.float32)
      return carry

    lax.fori_loop(0, C, zear, 0)

    @pl.when(c == 0)
    def _():
      pltpu.sync_copy(x_hbm.at[pl.ds(s * RPS, RPS)], s_sh.at[pl.ds(s * RPS, RPS)])

    @pl.when(c != 0)
    def _():
      _zero_stripe(s_sh, s * RPS, rows_v.at[0], D)

    _zero_stripe(t_sh, s * RPS, ea_v.at[0], DE)
    pltpu.sync_copy(zd_hbm.at[pl.ds(s * RPS, RPS)], d_sh.at[pl.ds(s * RPS, RPS)])
    pltpu.sync_copy(ei_hbm.at[0, pl.ds(wid * EPW, EPW)], src_v)
    pltpu.sync_copy(ones_hbm, ones_v)
    plsc.subcore_barrier()

    def batch(i0, nb):
      dds = [pltpu.async_copy(ei_hbm.at[1, pl.ds((wid * CPW + i0 + b) * C, C)],
                              dst_v.at[b], dsem) for b in range(nb)]
      rds = [pltpu.async_copy(x_hbm.at[src_v.at[pl.ds((i0 + b) * C, C)]],
                              rows_v.at[b], rsems[b]) for b in range(nb)]
      eds = [pltpu.async_copy(ea_hbm.at[pl.ds((wid * CPW + i0 + b) * C, C)],
                              ea_v.at[b], esems[b]) for b in range(nb)]
      for dd in dds:
        dd.wait()
      sds = []
      for b in range(nb):
        rds[b].wait()
        sds.append(pltpu.async_copy(rows_v.at[b], s_sh.at[dst_v.at[b]],
                                    ssems[b], add=True))
        eds[b].wait()
        sds.append(pltpu.async_copy(ea_v.at[b], t_sh.at[dst_v.at[b]],
                                    tsems[b], add=True))
        pltpu.sync_copy(ones_v, d_sh.at[dst_v.at[b]], add=True)
      for sd in sds:
        sd.wait()

    def outer(i, carry):
      batch(i * MB, MB)
      return carry

    lax.fori_loop(0, CPW // MB, outer, 0)
    if CPW % MB:
      batch(CPW - CPW % MB, CPW % MB)

    plsc.subcore_barrier()
    pltpu.sync_copy(s_sh.at[pl.ds(s * RPS, RPS)], s_out.at[c, pl.ds(s * RPS, RPS)])
    pltpu.sync_copy(t_sh.at[pl.ds(s * RPS, RPS)], t_out.at[c, pl.ds(s * RPS, RPS)])
    pltpu.sync_copy(d_sh.at[pl.ds(s * RPS, RPS)], d_out.at[c, pl.ds(s * RPS, RPS)])

  fn = pl.kernel(
      body,
      out_type=[
          jax.ShapeDtypeStruct((NC, N, D), jnp.float32),
          jax.ShapeDtypeStruct((NC, N, DE), jnp.float32),
          jax.ShapeDtypeStruct((NC, N, 8), jnp.float32),
      ],
      mesh=_mesh,
      compiler_params=_sc_params,
      scratch_types=[
          pltpu.VMEM((EPW,), jnp.int32),
          pltpu.VMEM((MB, C), jnp.int32),
          pltpu.VMEM((MB, C, D), jnp.float32),
          pltpu.VMEM((MB, C, DE), jnp.float32),
          pltpu.VMEM((C, 8), jnp.float32),
          pltpu.SemaphoreType.DMA,
          pltpu.SemaphoreType.DMA,
          pltpu.SemaphoreType.DMA,
          pltpu.SemaphoreType.DMA,
          pltpu.SemaphoreType.DMA,
          pltpu.SemaphoreType.DMA,
          pltpu.SemaphoreType.DMA,
          pltpu.SemaphoreType.DMA,
          pltpu.SemaphoreType.DMA,
          pltpu.VMEM_SHARED((N, D), jnp.float32),
          pltpu.VMEM_SHARED((N, DE), jnp.float32),
          pltpu.VMEM_SHARED((N, 8), jnp.float32),
      ],
  )
  return fn(x, ei, ea, zd, ones)


def _sc_spmm(tbl, ei):
  def body(tbl_hbm, ei_hbm, s_out,
           src_v, dst_v, rows_v, rsem0, rsem1, rsem2,
           ssem0, ssem1, ssem2, dsem, s_sh):
    c = lax.axis_index("c")
    s = lax.axis_index("s")
    wid = c * NS + s
    rsems = (rsem0, rsem1, rsem2)
    ssems = (ssem0, ssem1, ssem2)

    _zero_rows(rows_v.at[0], C)

    @pl.when(c == 0)
    def _():
      pltpu.sync_copy(tbl_hbm.at[pl.ds(s * RPS, RPS)], s_sh.at[pl.ds(s * RPS, RPS)])

    @pl.when(c != 0)
    def _():
      _zero_stripe(s_sh, s * RPS, rows_v.at[0], D)

    pltpu.sync_copy(ei_hbm.at[0, pl.ds(wid * EPW, EPW)], src_v)
    plsc.subcore_barrier()

    def batch(i0, nb):
      dds = [pltpu.async_copy(ei_hbm.at[1, pl.ds((wid * CPW + i0 + b) * C, C)],
                              dst_v.at[b], dsem) for b in range(nb)]
      rds = [pltpu.async_copy(tbl_hbm.at[src_v.at[pl.ds((i0 + b) * C, C)]],
                              rows_v.at[b], rsems[b]) for b in range(nb)]
      for dd in dds:
        dd.wait()
      sds = []
      for b in range(nb):
        rds[b].wait()
        sds.append(pltpu.async_copy(rows_v.at[b], s_sh.at[dst_v.at[b]],
                                    ssems[b], add=True))
      for sd in sds:
        sd.wait()

    def outer(i, carry):
      batch(i * NBUF, NBUF)
      return carry

    lax.fori_loop(0, CPW // NBUF, outer, 0)
    if CPW % NBUF:
      batch(CPW - CPW % NBUF, CPW % NBUF)

    plsc.subcore_barrier()
    pltpu.sync_copy(s_sh.at[pl.ds(s * RPS, RPS)], s_out.at[c, pl.ds(s * RPS, RPS)])

  fn = pl.kernel(
      body,
      out_type=jax.ShapeDtypeStruct((NC, N, D), jnp.float32),
      mesh=_mesh,
      compiler_params=_sc_params,
      scratch_types=[
          pltpu.VMEM((EPW,), jnp.int32),
          pltpu.VMEM((NBUF, C), jnp.int32),
          pltpu.VMEM((NBUF, C, D), jnp.float32),
          pltpu.SemaphoreType.DMA,
          pltpu.SemaphoreType.DMA,
          pltpu.SemaphoreType.DMA,
          pltpu.SemaphoreType.DMA,
          pltpu.SemaphoreType.DMA,
          pltpu.SemaphoreType.DMA,
          pltpu.SemaphoreType.DMA,
          pltpu.VMEM_SHARED((N, D), jnp.float32),
      ],
  )
  return fn(tbl, ei)


def _tc_body(s_ref, t_ref, d_ref, lwt_ref, ewt_ref, lbeb_ref, lb_ref,
             bias_ref, g_ref, b_ref, o_ref):
  a = s_ref[0] + s_ref[1]
  tt = t_ref[0] + t_ref[1]
  deg = (d_ref[0] + d_ref[1])[:, 0:1]
  aggr = jnp.dot(a, lwt_ref[...], preferred_element_type=jnp.float32)
  aggr = aggr + jnp.dot(tt, ewt_ref[...], preferred_element_type=jnp.float32)
  aggr = aggr + deg * lbeb_ref[...] + lb_ref[...]
  r = jnp.maximum(aggr, 0.0) + bias_ref[...]
  m = jnp.mean(r, axis=0, keepdims=True)
  cen = r - m
  v = jnp.mean(cen * cen, axis=0, keepdims=True)
  o_ref[...] = cen * lax.rsqrt(v + 1e-5) * g_ref[...] + b_ref[...]


def _tc_layer(sp, tp, dp, lw, lb, ew, eb, bias, g, b):
  lwt = lw.T
  ewt = ew.T
  lbeb = (lb + eb).reshape(1, H)
  return pl.pallas_call(
      _tc_body,
      out_shape=jax.ShapeDtypeStruct((N, H), jnp.float32),
  )(sp, tp, dp, lwt, ewt, lbeb, lb.reshape(1, H), bias.reshape(1, H),
    g.reshape(1, H), b.reshape(1, H))


def kernel(x, edge_index, edge_attr, lin1_w, lin1_b, edge1_w, edge1_b, bias1,
           bn1_g, bn1_b, lin2_w, lin2_b, edge2_w, edge2_b, bias2, bn2_g, bn2_b):
  ei = edge_index.astype(jnp.int32)
  zd = jnp.zeros((N, 8), jnp.float32)
  ones = jnp.ones((C, 8), jnp.float32)

  s1p, tp, dp = _sc_pass1(x, ei, edge_attr, zd, ones)
  h1 = _tc_layer(s1p, tp, dp, lin1_w, lin1_b, edge1_w, edge1_b, bias1,
                 bn1_g, bn1_b)
  s2p = _sc_spmm(h1, ei)
  out = _tc_layer(s2p, tp, dp, lin2_w, lin2_b, edge2_w, edge2_b, bias2,
                  bn2_g, bn2_b)
  return out


# speedup vs baseline: 9.7900x; 1.0094x over previous
import jax
import jax.numpy as jnp
from jax import lax
from jax.experimental import pallas as pl
from jax.experimental.pallas import tpu as pltpu
from jax.experimental.pallas import tpu_sc as plsc

N = 10000
E = 320000
D = 128
DE = 16
H = 128

NC = 2
NS = 16
NW = NC * NS
C = 80
NCHUNKS = E // C
CPW = NCHUNKS // NW
RPS = N // NS
NBUF = 3
EBUF = 5

_mesh = plsc.VectorSubcoreMesh(core_axis_name="c", subcore_axis_name="s")
_sc_params = pltpu.CompilerParams(use_tc_tiling_on_sc=False)


MB = 2


EPW = E // NW


def _zero_rows(buf, nrows):
  zv = jnp.zeros((16,), jnp.float32)

  def zrow(r, carry):
    for k in range(8):
      buf[r, pl.ds(k * 16, 16)] = zv
    return carry

  lax.fori_loop(0, nrows, zrow, 0)


def _zero_stripe(sh, base, zsrc, width):
  for k in range(RPS // C):
    pltpu.sync_copy(zsrc, sh.at[pl.ds(base + k * C, C)])
  rem = RPS % C
  if rem:
    pltpu.sync_copy(zsrc.at[pl.ds(0, rem)],
                    sh.at[pl.ds(base + (RPS // C) * C, rem)])


def _sc_pass1(x, ei, ea, zd, ones):
  def body(x_hbm, ei_hbm, ea_hbm, zd_hbm, ones_hbm,
           s_out, t_out, d_out,
           src_v, dst_v, rows_v, ea_v, ones_v,
           rsem0, rsem1, ssem0, ssem1, esem0, esem1, tsem0, tsem1,
           osem0, osem1, dsem,
           s_sh, t_sh, d_sh):
    c = lax.axis_index("c")
    s = lax.axis_index("s")
    wid = c * NS + s
    rsems = (rsem0, rsem1)
    ssems = (ssem0, ssem1)
    esems = (esem0, esem1)
    tsems = (tsem0, tsem1)
    osems = (osem0, osem1)

    _zero_rows(rows_v.at[0], C)

    def zear(r, carry):
      ea_v[0, r, :] = jnp.zeros((16,), jnp.float32)
      return carry

    lax.fori_loop(0, C, zear, 0)

    @pl.when(c == 0)
    def _():
      pltpu.sync_copy(x_hbm.at[pl.ds(s * RPS, RPS)], s_sh.at[pl.ds(s * RPS, RPS)])

    @pl.when(c != 0)
    def _():
      _zero_stripe(s_sh, s * RPS, rows_v.at[0], D)

    _zero_stripe(t_sh, s * RPS, ea_v.at[0], DE)
    pltpu.sync_copy(zd_hbm.at[pl.ds(s * RPS, RPS)], d_sh.at[pl.ds(s * RPS, RPS)])
    pltpu.sync_copy(ei_hbm.at[0, pl.ds(wid * EPW, EPW)], src_v)
    pltpu.sync_copy(ones_hbm, ones_v)
    plsc.subcore_barrier()

    def batch(i0, nb):
      dds = [pltpu.async_copy(ei_hbm.at[1, pl.ds((wid * CPW + i0 + b) * C, C)],
                              dst_v.at[b], dsem) for b in range(nb)]
      rds = [pltpu.async_copy(x_hbm.at[src_v.at[pl.ds((i0 + b) * C, C)]],
                              rows_v.at[b], rsems[b]) for b in range(nb)]
      eds = [pltpu.async_copy(ea_hbm.at[pl.ds((wid * CPW + i0 + b) * C, C)],
                              ea_v.at[b], esems[b]) for b in range(nb)]
      for dd in dds:
        dd.wait()
      sds = []
      for b in range(nb):
        rds[b].wait()
        sds.append(pltpu.async_copy(rows_v.at[b], s_sh.at[dst_v.at[b]],
                                    ssems[b], add=True))
        eds[b].wait()
        sds.append(pltpu.async_copy(ea_v.at[b], t_sh.at[dst_v.at[b]],
                                    tsems[b], add=True))
        sds.append(pltpu.async_copy(ones_v, d_sh.at[dst_v.at[b]],
                                    osems[b], add=True))
      for sd in sds:
        sd.wait()

    def outer(i, carry):
      batch(i * MB, MB)
      return carry

    lax.fori_loop(0, CPW // MB, outer, 0)
    if CPW % MB:
      batch(CPW - CPW % MB, CPW % MB)

    plsc.subcore_barrier()
    pltpu.sync_copy(s_sh.at[pl.ds(s * RPS, RPS)], s_out.at[c, pl.ds(s * RPS, RPS)])
    pltpu.sync_copy(t_sh.at[pl.ds(s * RPS, RPS)], t_out.at[c, pl.ds(s * RPS, RPS)])
    pltpu.sync_copy(d_sh.at[pl.ds(s * RPS, RPS)], d_out.at[c, pl.ds(s * RPS, RPS)])

  fn = pl.kernel(
      body,
      out_type=[
          jax.ShapeDtypeStruct((NC, N, D), jnp.float32),
          jax.ShapeDtypeStruct((NC, N, DE), jnp.float32),
          jax.ShapeDtypeStruct((NC, N, 8), jnp.float32),
      ],
      mesh=_mesh,
      compiler_params=_sc_params,
      scratch_types=[
          pltpu.VMEM((EPW,), jnp.int32),
          pltpu.VMEM((MB, C), jnp.int32),
          pltpu.VMEM((MB, C, D), jnp.float32),
          pltpu.VMEM((MB, C, DE), jnp.float32),
          pltpu.VMEM((C, 8), jnp.float32),
          pltpu.SemaphoreType.DMA,
          pltpu.SemaphoreType.DMA,
          pltpu.SemaphoreType.DMA,
          pltpu.SemaphoreType.DMA,
          pltpu.SemaphoreType.DMA,
          pltpu.SemaphoreType.DMA,
          pltpu.SemaphoreType.DMA,
          pltpu.SemaphoreType.DMA,
          pltpu.SemaphoreType.DMA,
          pltpu.SemaphoreType.DMA,
          pltpu.SemaphoreType.DMA,
          pltpu.VMEM_SHARED((N, D), jnp.float32),
          pltpu.VMEM_SHARED((N, DE), jnp.float32),
          pltpu.VMEM_SHARED((N, 8), jnp.float32),
      ],
  )
  return fn(x, ei, ea, zd, ones)


def _sc_spmm(tbl, ei):
  def body(tbl_hbm, ei_hbm, s_out,
           src_v, dst_v, rows_v, rsem0, rsem1, rsem2,
           ssem0, ssem1, ssem2, dsem, s_sh):
    c = lax.axis_index("c")
    s = lax.axis_index("s")
    wid = c * NS + s
    rsems = (rsem0, rsem1, rsem2)
    ssems = (ssem0, ssem1, ssem2)

    _zero_rows(rows_v.at[0], C)

    @pl.when(c == 0)
    def _():
      pltpu.sync_copy(tbl_hbm.at[pl.ds(s * RPS, RPS)], s_sh.at[pl.ds(s * RPS, RPS)])

    @pl.when(c != 0)
    def _():
      _zero_stripe(s_sh, s * RPS, rows_v.at[0], D)

    pltpu.sync_copy(ei_hbm.at[0, pl.ds(wid * EPW, EPW)], src_v)
    plsc.subcore_barrier()

    def batch(i0, nb):
      dds = [pltpu.async_copy(ei_hbm.at[1, pl.ds((wid * CPW + i0 + b) * C, C)],
                              dst_v.at[b], dsem) for b in range(nb)]
      rds = [pltpu.async_copy(tbl_hbm.at[src_v.at[pl.ds((i0 + b) * C, C)]],
                              rows_v.at[b], rsems[b]) for b in range(nb)]
      for dd in dds:
        dd.wait()
      sds = []
      for b in range(nb):
        rds[b].wait()
        sds.append(pltpu.async_copy(rows_v.at[b], s_sh.at[dst_v.at[b]],
                                    ssems[b], add=True))
      for sd in sds:
        sd.wait()

    def outer(i, carry):
      batch(i * NBUF, NBUF)
      return carry

    lax.fori_loop(0, CPW // NBUF, outer, 0)
    if CPW % NBUF:
      batch(CPW - CPW % NBUF, CPW % NBUF)

    plsc.subcore_barrier()
    pltpu.sync_copy(s_sh.at[pl.ds(s * RPS, RPS)], s_out.at[c, pl.ds(s * RPS, RPS)])

  fn = pl.kernel(
      body,
      out_type=jax.ShapeDtypeStruct((NC, N, D), jnp.float32),
      mesh=_mesh,
      compiler_params=_sc_params,
      scratch_types=[
          pltpu.VMEM((EPW,), jnp.int32),
          pltpu.VMEM((NBUF, C), jnp.int32),
          pltpu.VMEM((NBUF, C, D), jnp.float32),
          pltpu.SemaphoreType.DMA,
          pltpu.SemaphoreType.DMA,
          pltpu.SemaphoreType.DMA,
          pltpu.SemaphoreType.DMA,
          pltpu.SemaphoreType.DMA,
          pltpu.SemaphoreType.DMA,
          pltpu.SemaphoreType.DMA,
          pltpu.VMEM_SHARED((N, D), jnp.float32),
      ],
  )
  return fn(tbl, ei)


def _tc_body(s_ref, t_ref, d_ref, lwt_ref, ewt_ref, lbeb_ref, lb_ref,
             bias_ref, g_ref, b_ref, o_ref):
  a = s_ref[0] + s_ref[1]
  tt = t_ref[0] + t_ref[1]
  deg = (d_ref[0] + d_ref[1])[:, 0:1]
  aggr = jnp.dot(a, lwt_ref[...], preferred_element_type=jnp.float32)
  aggr = aggr + jnp.dot(tt, ewt_ref[...], preferred_element_type=jnp.float32)
  aggr = aggr + deg * lbeb_ref[...] + lb_ref[...]
  r = jnp.maximum(aggr, 0.0) + bias_ref[...]
  m = jnp.mean(r, axis=0, keepdims=True)
  cen = r - m
  v = jnp.mean(cen * cen, axis=0, keepdims=True)
  o_ref[...] = cen * lax.rsqrt(v + 1e-5) * g_ref[...] + b_ref[...]


def _tc_layer(sp, tp, dp, lw, lb, ew, eb, bias, g, b):
  lwt = lw.T
  ewt = ew.T
  lbeb = (lb + eb).reshape(1, H)
  return pl.pallas_call(
      _tc_body,
      out_shape=jax.ShapeDtypeStruct((N, H), jnp.float32),
  )(sp, tp, dp, lwt, ewt, lbeb, lb.reshape(1, H), bias.reshape(1, H),
    g.reshape(1, H), b.reshape(1, H))


def kernel(x, edge_index, edge_attr, lin1_w, lin1_b, edge1_w, edge1_b, bias1,
           bn1_g, bn1_b, lin2_w, lin2_b, edge2_w, edge2_b, bias2, bn2_g, bn2_b):
  ei = edge_index.astype(jnp.int32)
  zd = jnp.zeros((N, 8), jnp.float32)
  ones = jnp.ones((C, 8), jnp.float32)

  s1p, tp, dp = _sc_pass1(x, ei, edge_attr, zd, ones)
  h1 = _tc_layer(s1p, tp, dp, lin1_w, lin1_b, edge1_w, edge1_b, bias1,
                 bn1_g, bn1_b)
  s2p = _sc_spmm(h1, ei)
  out = _tc_layer(s2p, tp, dp, lin2_w, lin2_b, edge2_w, edge2_b, bias2,
                  bn2_g, bn2_b)
  return out
